# Initial kernel scaffold; baseline (speedup 1.0000x reference)
import jax
import jax.numpy as jnp
from jax.experimental import pallas as pl


def kernel(x, weight_t, bias):
    raise NotImplementedError("write your pallas kernel here")


# speedup vs baseline: 1.7446x; 1.7446x over previous
import jax
import jax.numpy as jnp
from jax.experimental import pallas as pl

_MODE = 0.02
_THRESHOLD = 0.1

_BK = 512


def _matvec_body(x_ref, w_ref, b_ref, o_ref):
    i = pl.program_id(0)
    xb = x_ref[...]
    xm = xb - _MODE
    v = jnp.where(jnp.abs(xm) > _THRESHOLD, xm, 0.0) + _MODE
    partial = jnp.sum(w_ref[...] * v, axis=0, keepdims=True)

    @pl.when(i == 0)
    def _():
        o_ref[...] = b_ref[...] + partial

    @pl.when(i > 0)
    def _():
        o_ref[...] += partial


def kernel(x, weight_t, bias):
    k, n = weight_t.shape
    xa = x.reshape(k, 1)
    b2 = bias.reshape(1, n)
    out = pl.pallas_call(
        _matvec_body,
        grid=(k // _BK,),
        in_specs=[
            pl.BlockSpec((_BK, 1), lambda i: (i, 0)),
            pl.BlockSpec((_BK, n), lambda i: (i, 0)),
            pl.BlockSpec((1, n), lambda i: (0, 0)),
        ],
        out_specs=pl.BlockSpec((1, n), lambda i: (0, 0)),
        out_shape=jax.ShapeDtypeStruct((1, n), jnp.float32),
    )(xa, weight_t, b2)
    return out
